# Initial kernel scaffold; baseline (speedup 1.0000x reference)
import jax
import jax.numpy as jnp
from jax.experimental import pallas as pl


def kernel(x, edge_index, edge_weight, enc_W1, enc_b1, enc_W2, enc_b2, e1_We, e1_be, g1_W1, g1_b1, g1_W2, g1_b2, e2_We, e2_be, g2_W1, g2_b1, g2_W2, g2_b2, post_W, post_b):
    raise NotImplementedError("write your pallas kernel here")


# speedup vs baseline: 8.4503x; 8.4503x over previous
import functools

import jax
import jax.numpy as jnp
from jax import lax
from jax.experimental import pallas as pl
from jax.experimental.pallas import tpu as pltpu
from jax.experimental.pallas import tpu_sc as plsc

_NC = 2
_NS = 16
_CHUNK = 128
_BISECT_ITERS = 48


def _topk_mask_block(y, k):
    kf = jnp.float32(k)
    lo0 = jnp.min(y, axis=1, keepdims=True) - 1.0
    hi0 = jnp.max(y, axis=1, keepdims=True)

    def body(_, carry):
        lo, hi = carry
        mid = 0.5 * (lo + hi)
        cnt = jnp.sum((y > mid).astype(jnp.float32), axis=1, keepdims=True)
        pred = cnt >= kf
        return jnp.where(pred, mid, lo), jnp.where(pred, hi, mid)

    lo, _ = lax.fori_loop(0, _BISECT_ITERS, body, (lo0, hi0))
    return jnp.where(y > lo, jnp.maximum(y, 0.0), 0.0)


def _leaky(t):
    return jnp.where(t >= 0.0, t, 0.01 * t)


def _enc_body(x_ref, w1_ref, b1_ref, w2_ref, b2_ref, o_ref, *, k):
    h = jnp.dot(x_ref[...], w1_ref[...], preferred_element_type=jnp.float32)
    h = _leaky(h + b1_ref[...])
    y = jnp.dot(h, w2_ref[...], preferred_element_type=jnp.float32) + b2_ref[...]
    o_ref[...] = _topk_mask_block(y, k)


def _gine_mlp_body(h_ref, p0_ref, p1_ref, w1_ref, b1_ref, w2_ref, b2_ref,
                   o_ref, *, k):
    g = h_ref[...] + p0_ref[...] + p1_ref[...]
    t = jnp.dot(g, w1_ref[...], preferred_element_type=jnp.float32)
    t = _leaky(t + b1_ref[...])
    y = jnp.dot(t, w2_ref[...], preferred_element_type=jnp.float32) + b2_ref[...]
    o_ref[...] = h_ref[...] + _topk_mask_block(y, k)


def _post_body(h_ref, w_ref, b_ref, o_ref, *, k):
    y = jnp.dot(h_ref[...], w_ref[...], preferred_element_type=jnp.float32)
    o_ref[...] = _topk_mask_block(y + b_ref[...], k)


def _edge_body(ew_ref, we1_ref, be1_ref, we2_ref, be2_ref, e1_ref, e2_ref):
    ew = ew_ref[...]
    e1_ref[...] = jnp.dot(ew, we1_ref[...], preferred_element_type=jnp.float32) + be1_ref[...]
    e2_ref[...] = jnp.dot(ew, we2_ref[...], preferred_element_type=jnp.float32) + be2_ref[...]


def _full_spec(shape):
    return pl.BlockSpec(shape, lambda i: (0,) * len(shape))


def _enc_call(x, w1, b1, w2, b2, k):
    n, cin = x.shape
    hid = w1.shape[1]
    emb = w2.shape[1]
    r = 1000
    return pl.pallas_call(
        functools.partial(_enc_body, k=k),
        grid=(n // r,),
        in_specs=[
            pl.BlockSpec((r, cin), lambda i: (i, 0)),
            _full_spec((cin, hid)), _full_spec((1, hid)),
            _full_spec((hid, emb)), _full_spec((1, emb)),
        ],
        out_specs=pl.BlockSpec((r, emb), lambda i: (i, 0)),
        out_shape=jax.ShapeDtypeStruct((n, emb), jnp.float32),
    )(x, w1, b1, w2, b2)


def _gine_mlp_call(h, p0, p1, w1, b1, w2, b2, k):
    n, emb = h.shape
    hid = w1.shape[1]
    r = 1000
    blk = pl.BlockSpec((r, emb), lambda i: (i, 0))
    return pl.pallas_call(
        functools.partial(_gine_mlp_body, k=k),
        grid=(n // r,),
        in_specs=[
            blk, blk, blk,
            _full_spec((emb, hid)), _full_spec((1, hid)),
            _full_spec((hid, emb)), _full_spec((1, emb)),
        ],
        out_specs=blk,
        out_shape=jax.ShapeDtypeStruct((n, emb), jnp.float32),
    )(h, p0, p1, w1, b1, w2, b2)


def _post_call(h, w, b, k):
    n, emb = h.shape
    cout = w.shape[1]
    r = 1000
    return pl.pallas_call(
        functools.partial(_post_body, k=k),
        grid=(n // r,),
        in_specs=[
            pl.BlockSpec((r, emb), lambda i: (i, 0)),
            _full_spec((emb, cout)), _full_spec((1, cout)),
        ],
        out_specs=pl.BlockSpec((r, cout), lambda i: (i, 0)),
        out_shape=jax.ShapeDtypeStruct((n, cout), jnp.float32),
    )(h, w, b)


def _edge_call(ew, we1, be1, we2, be2):
    e, ed = ew.shape
    emb = we1.shape[1]
    r = 4000
    return pl.pallas_call(
        _edge_body,
        grid=(e // r,),
        in_specs=[
            pl.BlockSpec((r, ed), lambda i: (i, 0)),
            _full_spec((ed, emb)), _full_spec((1, emb)),
            _full_spec((ed, emb)), _full_spec((1, emb)),
        ],
        out_specs=[pl.BlockSpec((r, emb), lambda i: (i, 0))] * 2,
        out_shape=[jax.ShapeDtypeStruct((e, emb), jnp.float32)] * 2,
    )(ew, we1, be1, we2, be2)


def _make_seg_kernel(n, e, emb):
    nchunks_total = e // _CHUNK
    chunks_per_core = nchunks_total // _NC
    zrows = (n // _NS) & ~7
    tail = n - _NS * zrows
    nvec = emb // 16
    mesh = plsc.VectorSubcoreMesh(core_axis_name="c", subcore_axis_name="s")

    @functools.partial(
        pl.kernel,
        out_type=jax.ShapeDtypeStruct((_NC, n, emb), jnp.float32),
        mesh=mesh,
        compiler_params=pltpu.CompilerParams(use_tc_tiling_on_sc=False),
        scratch_types=[
            pltpu.VMEM((_CHUNK,), jnp.int32),
            pltpu.VMEM((_CHUNK,), jnp.int32),
            pltpu.VMEM((_CHUNK, emb), jnp.float32),
            pltpu.VMEM((_CHUNK, emb), jnp.float32),
            pltpu.VMEM((zrows, emb), jnp.float32),
            pltpu.VMEM_SHARED((n, emb), jnp.float32),
            pltpu.SemaphoreType.DMA,
        ],
    )
    def seg(h_hbm, e_hbm, src_hbm, dst_hbm, out_hbm,
            src_v, dst_v, rows_v, e_v, z_v, aggr_sh, sem):
        c = lax.axis_index("c")
        s = lax.axis_index("s")

        def zbody(r2, _):
            for v in range(nvec):
                z_v[r2, pl.ds(v * 16, 16)] = jnp.zeros((16,), jnp.float32)
            return 0
        lax.fori_loop(0, zrows, zbody, 0)
        pltpu.sync_copy(z_v, aggr_sh.at[pl.ds(s * zrows, zrows)])

        @pl.when(s == _NS - 1)
        def _():
            pltpu.sync_copy(z_v.at[pl.ds(0, tail)],
                            aggr_sh.at[pl.ds(_NS * zrows, tail)])
        plsc.subcore_barrier()

        nch = jnp.where(s < (chunks_per_core % _NS), chunks_per_core // _NS + 1,
                        chunks_per_core // _NS)

        def chunk_body(m, _):
            q = c * chunks_per_core + s + _NS * m
            base = q * _CHUNK
            pltpu.sync_copy(src_hbm.at[pl.ds(base, _CHUNK)], src_v)
            pltpu.sync_copy(dst_hbm.at[pl.ds(base, _CHUNK)], dst_v)
            cp = pltpu.async_copy(h_hbm.at[src_v], rows_v, sem)
            pltpu.sync_copy(e_hbm.at[pl.ds(base, _CHUNK)], e_v)
            cp.wait()

            def rbody(r2, _):
                for v in range(nvec):
                    sl = pl.ds(v * 16, 16)
                    rows_v[r2, sl] = jnp.maximum(rows_v[r2, sl] + e_v[r2, sl], 0.0)
                return 0
            lax.fori_loop(0, _CHUNK, rbody, 0)
            pltpu.sync_copy(rows_v, aggr_sh.at[dst_v], add=True)
            return 0
        lax.fori_loop(0, nch, chunk_body, 0)

        plsc.subcore_barrier()
        pltpu.sync_copy(aggr_sh.at[pl.ds(s * zrows, zrows)],
                        out_hbm.at[c, pl.ds(s * zrows, zrows)])

        @pl.when(s == _NS - 1)
        def _():
            pltpu.sync_copy(aggr_sh.at[pl.ds(_NS * zrows, tail)],
                            out_hbm.at[c, pl.ds(_NS * zrows, tail)])

    return seg


def kernel(x, edge_index, edge_weight,
           enc_W1, enc_b1, enc_W2, enc_b2,
           e1_We, e1_be, g1_W1, g1_b1, g1_W2, g1_b2,
           e2_We, e2_be, g2_W1, g2_b1, g2_W2, g2_b2,
           post_W, post_b):
    n = x.shape[0]
    e = edge_weight.shape[0]
    emb = enc_W2.shape[1]
    tk = emb // 2
    src = edge_index[0]
    dst = edge_index[1]

    h0 = _enc_call(x, enc_W1, enc_b1.reshape(1, -1), enc_W2,
                   enc_b2.reshape(1, -1), tk)
    e1, e2 = _edge_call(edge_weight, e1_We, e1_be.reshape(1, -1),
                        e2_We, e2_be.reshape(1, -1))
    seg = _make_seg_kernel(n, e, emb)

    p1 = seg(h0, e1, src, dst)
    h1 = _gine_mlp_call(h0, p1[0], p1[1], g1_W1, g1_b1.reshape(1, -1),
                        g1_W2, g1_b2.reshape(1, -1), tk)
    p2 = seg(h1, e2, src, dst)
    h2 = _gine_mlp_call(h1, p2[0], p2[1], g2_W1, g2_b1.reshape(1, -1),
                        g2_W2, g2_b2.reshape(1, -1), tk)
    return _post_call(h2, post_W, post_b.reshape(1, -1), post_W.shape[1] // 2)


# speedup vs baseline: 15.1583x; 1.7938x over previous
import functools

import jax
import jax.numpy as jnp
from jax import lax
from jax.experimental import pallas as pl
from jax.experimental.pallas import tpu as pltpu
from jax.experimental.pallas import tpu_sc as plsc

_NC = 2
_NS = 16
_CHUNK = 128
_BISECT_ITERS = 40


def _topk_mask_block(y, k):
    kf = jnp.float32(k)
    yt = y.T
    lo0 = jnp.min(yt, axis=0, keepdims=True) - 1.0
    hi0 = jnp.max(yt, axis=0, keepdims=True)

    def body(_, carry):
        lo, hi = carry
        mid = 0.5 * (lo + hi)
        cnt = jnp.sum((yt > mid).astype(jnp.float32), axis=0, keepdims=True)
        pred = cnt >= kf
        return jnp.where(pred, mid, lo), jnp.where(pred, hi, mid)

    lo, _ = lax.fori_loop(0, _BISECT_ITERS, body, (lo0, hi0))
    return jnp.where(yt > lo, jnp.maximum(yt, 0.0), 0.0).T


def _leaky(t):
    return jnp.where(t >= 0.0, t, 0.01 * t)


def _enc_body(x_ref, w1_ref, b1_ref, w2_ref, b2_ref, o_ref, *, k):
    h = jnp.dot(x_ref[...], w1_ref[...], preferred_element_type=jnp.float32)
    h = _leaky(h + b1_ref[...])
    y = jnp.dot(h, w2_ref[...], preferred_element_type=jnp.float32) + b2_ref[...]
    o_ref[...] = _topk_mask_block(y, k)


def _gine_mlp_body(h_ref, p0_ref, p1_ref, w1_ref, b1_ref, w2_ref, b2_ref,
                   o_ref, *, k):
    g = h_ref[...] + p0_ref[...] + p1_ref[...]
    t = jnp.dot(g, w1_ref[...], preferred_element_type=jnp.float32)
    t = _leaky(t + b1_ref[...])
    y = jnp.dot(t, w2_ref[...], preferred_element_type=jnp.float32) + b2_ref[...]
    o_ref[...] = h_ref[...] + _topk_mask_block(y, k)


def _post_body(h_ref, w_ref, b_ref, o_ref, *, k):
    y = jnp.dot(h_ref[...], w_ref[...], preferred_element_type=jnp.float32)
    o_ref[...] = _topk_mask_block(y + b_ref[...], k)


def _edge_body(ew_ref, we1_ref, be1_ref, we2_ref, be2_ref, e1_ref, e2_ref):
    ew = ew_ref[...]
    e1_ref[...] = jnp.dot(ew, we1_ref[...], preferred_element_type=jnp.float32) + be1_ref[...]
    e2_ref[...] = jnp.dot(ew, we2_ref[...], preferred_element_type=jnp.float32) + be2_ref[...]


def _full_spec(shape):
    return pl.BlockSpec(shape, lambda i: (0,) * len(shape))


def _enc_call(x, w1, b1, w2, b2, k):
    n, cin = x.shape
    hid = w1.shape[1]
    emb = w2.shape[1]
    r = 1000
    return pl.pallas_call(
        functools.partial(_enc_body, k=k),
        grid=(n // r,),
        in_specs=[
            pl.BlockSpec((r, cin), lambda i: (i, 0)),
            _full_spec((cin, hid)), _full_spec((1, hid)),
            _full_spec((hid, emb)), _full_spec((1, emb)),
        ],
        out_specs=pl.BlockSpec((r, emb), lambda i: (i, 0)),
        out_shape=jax.ShapeDtypeStruct((n, emb), jnp.float32),
    )(x, w1, b1, w2, b2)


def _gine_mlp_call(h, p0, p1, w1, b1, w2, b2, k):
    n, emb = h.shape
    hid = w1.shape[1]
    r = 1000
    blk = pl.BlockSpec((r, emb), lambda i: (i, 0))
    return pl.pallas_call(
        functools.partial(_gine_mlp_body, k=k),
        grid=(n // r,),
        in_specs=[
            blk, blk, blk,
            _full_spec((emb, hid)), _full_spec((1, hid)),
            _full_spec((hid, emb)), _full_spec((1, emb)),
        ],
        out_specs=blk,
        out_shape=jax.ShapeDtypeStruct((n, emb), jnp.float32),
    )(h, p0, p1, w1, b1, w2, b2)


def _post_call(h, w, b, k):
    n, emb = h.shape
    cout = w.shape[1]
    r = 1000
    return pl.pallas_call(
        functools.partial(_post_body, k=k),
        grid=(n // r,),
        in_specs=[
            pl.BlockSpec((r, emb), lambda i: (i, 0)),
            _full_spec((emb, cout)), _full_spec((1, cout)),
        ],
        out_specs=pl.BlockSpec((r, cout), lambda i: (i, 0)),
        out_shape=jax.ShapeDtypeStruct((n, cout), jnp.float32),
    )(h, w, b)


def _edge_call(ew, we1, be1, we2, be2):
    e, ed = ew.shape
    emb = we1.shape[1]
    r = 4000
    return pl.pallas_call(
        _edge_body,
        grid=(e // r,),
        in_specs=[
            pl.BlockSpec((r, ed), lambda i: (i, 0)),
            _full_spec((ed, emb)), _full_spec((1, emb)),
            _full_spec((ed, emb)), _full_spec((1, emb)),
        ],
        out_specs=[pl.BlockSpec((r, emb), lambda i: (i, 0))] * 2,
        out_shape=[jax.ShapeDtypeStruct((e, emb), jnp.float32)] * 2,
    )(ew, we1, be1, we2, be2)


def _make_seg_kernel(n, e, emb):
    nchunks_total = e // _CHUNK
    chunks_per_core = nchunks_total // _NC
    zrows = (n // _NS) & ~7
    tail = n - _NS * zrows
    nvec = emb // 16
    mesh = plsc.VectorSubcoreMesh(core_axis_name="c", subcore_axis_name="s")

    @functools.partial(
        pl.kernel,
        out_type=jax.ShapeDtypeStruct((_NC, n, emb), jnp.float32),
        mesh=mesh,
        compiler_params=pltpu.CompilerParams(use_tc_tiling_on_sc=False),
        scratch_types=[
            pltpu.VMEM((_CHUNK,), jnp.int32),
            pltpu.VMEM((_CHUNK,), jnp.int32),
            pltpu.VMEM((_CHUNK, emb), jnp.float32),
            pltpu.VMEM((_CHUNK, emb), jnp.float32),
            pltpu.VMEM((zrows, emb), jnp.float32),
            pltpu.VMEM_SHARED((n, emb), jnp.float32),
            pltpu.SemaphoreType.DMA,
        ],
    )
    def seg(h_hbm, e_hbm, src_hbm, dst_hbm, out_hbm,
            src_v, dst_v, rows_v, e_v, z_v, aggr_sh, sem):
        c = lax.axis_index("c")
        s = lax.axis_index("s")

        def zbody(r2, _):
            for v in range(nvec):
                z_v[r2, pl.ds(v * 16, 16)] = jnp.zeros((16,), jnp.float32)
            return 0
        lax.fori_loop(0, zrows, zbody, 0)
        pltpu.sync_copy(z_v, aggr_sh.at[pl.ds(s * zrows, zrows)])

        @pl.when(s == _NS - 1)
        def _():
            pltpu.sync_copy(z_v.at[pl.ds(0, tail)],
                            aggr_sh.at[pl.ds(_NS * zrows, tail)])
        plsc.subcore_barrier()

        nch = jnp.where(s < (chunks_per_core % _NS), chunks_per_core // _NS + 1,
                        chunks_per_core // _NS)

        def chunk_body(m, _):
            q = c * chunks_per_core + s + _NS * m
            base = q * _CHUNK
            pltpu.sync_copy(src_hbm.at[pl.ds(base, _CHUNK)], src_v)
            pltpu.sync_copy(dst_hbm.at[pl.ds(base, _CHUNK)], dst_v)
            cp = pltpu.async_copy(h_hbm.at[src_v], rows_v, sem)
            pltpu.sync_copy(e_hbm.at[pl.ds(base, _CHUNK)], e_v)
            cp.wait()

            def rbody(r2, _):
                for v in range(nvec):
                    sl = pl.ds(v * 16, 16)
                    rows_v[r2, sl] = jnp.maximum(rows_v[r2, sl] + e_v[r2, sl], 0.0)
                return 0
            lax.fori_loop(0, _CHUNK, rbody, 0)
            pltpu.sync_copy(rows_v, aggr_sh.at[dst_v], add=True)
            return 0
        lax.fori_loop(0, nch, chunk_body, 0)

        plsc.subcore_barrier()
        pltpu.sync_copy(aggr_sh.at[pl.ds(s * zrows, zrows)],
                        out_hbm.at[c, pl.ds(s * zrows, zrows)])

        @pl.when(s == _NS - 1)
        def _():
            pltpu.sync_copy(aggr_sh.at[pl.ds(_NS * zrows, tail)],
                            out_hbm.at[c, pl.ds(_NS * zrows, tail)])

    return seg


def kernel(x, edge_index, edge_weight,
           enc_W1, enc_b1, enc_W2, enc_b2,
           e1_We, e1_be, g1_W1, g1_b1, g1_W2, g1_b2,
           e2_We, e2_be, g2_W1, g2_b1, g2_W2, g2_b2,
           post_W, post_b):
    n = x.shape[0]
    e = edge_weight.shape[0]
    emb = enc_W2.shape[1]
    tk = emb // 2
    src = edge_index[0]
    dst = edge_index[1]

    h0 = _enc_call(x, enc_W1, enc_b1.reshape(1, -1), enc_W2,
                   enc_b2.reshape(1, -1), tk)
    e1, e2 = _edge_call(edge_weight, e1_We, e1_be.reshape(1, -1),
                        e2_We, e2_be.reshape(1, -1))
    seg = _make_seg_kernel(n, e, emb)

    p1 = seg(h0, e1, src, dst)
    h1 = _gine_mlp_call(h0, p1[0], p1[1], g1_W1, g1_b1.reshape(1, -1),
                        g1_W2, g1_b2.reshape(1, -1), tk)
    p2 = seg(h1, e2, src, dst)
    h2 = _gine_mlp_call(h1, p2[0], p2[1], g2_W1, g2_b1.reshape(1, -1),
                        g2_W2, g2_b2.reshape(1, -1), tk)
    return _post_call(h2, post_W, post_b.reshape(1, -1), post_W.shape[1] // 2)


# speedup vs baseline: 17.6399x; 1.1637x over previous
import functools

import jax
import jax.numpy as jnp
from jax import lax
from jax.experimental import pallas as pl
from jax.experimental.pallas import tpu as pltpu
from jax.experimental.pallas import tpu_sc as plsc

_NC = 2
_NS = 16
_CHUNK = 128
_BISECT_ITERS = 40


def _topk_mask_block(y, k):
    kf = jnp.float32(k)
    yt = y.T
    lo0 = jnp.min(yt, axis=0, keepdims=True) - 1.0
    hi0 = jnp.max(yt, axis=0, keepdims=True)

    def body(_, carry):
        lo, hi = carry
        mid = 0.5 * (lo + hi)
        cnt = jnp.sum((yt > mid).astype(jnp.float32), axis=0, keepdims=True)
        pred = cnt >= kf
        return jnp.where(pred, mid, lo), jnp.where(pred, hi, mid)

    lo, _ = lax.fori_loop(0, _BISECT_ITERS, body, (lo0, hi0))
    return jnp.where(yt > lo, jnp.maximum(yt, 0.0), 0.0).T


def _leaky(t):
    return jnp.where(t >= 0.0, t, 0.01 * t)


def _enc_body(x_ref, w1_ref, b1_ref, w2_ref, b2_ref, o_ref, *, k):
    h = jnp.dot(x_ref[...], w1_ref[...], preferred_element_type=jnp.float32)
    h = _leaky(h + b1_ref[...])
    y = jnp.dot(h, w2_ref[...], preferred_element_type=jnp.float32) + b2_ref[...]
    o_ref[...] = _topk_mask_block(y, k)


def _gine_mlp_body(h_ref, p0_ref, p1_ref, w1_ref, b1_ref, w2_ref, b2_ref,
                   o_ref, *, k):
    g = h_ref[...] + p0_ref[...] + p1_ref[...]
    t = jnp.dot(g, w1_ref[...], preferred_element_type=jnp.float32)
    t = _leaky(t + b1_ref[...])
    y = jnp.dot(t, w2_ref[...], preferred_element_type=jnp.float32) + b2_ref[...]
    o_ref[...] = h_ref[...] + _topk_mask_block(y, k)


def _post_body(h_ref, w_ref, b_ref, o_ref, *, k):
    y = jnp.dot(h_ref[...], w_ref[...], preferred_element_type=jnp.float32)
    o_ref[...] = _topk_mask_block(y + b_ref[...], k)


def _edge_body(ew_ref, we1_ref, be1_ref, we2_ref, be2_ref, e1_ref, e2_ref):
    ew = ew_ref[...]
    e1_ref[...] = jnp.dot(ew, we1_ref[...], preferred_element_type=jnp.float32) + be1_ref[...]
    e2_ref[...] = jnp.dot(ew, we2_ref[...], preferred_element_type=jnp.float32) + be2_ref[...]


def _full_spec(shape):
    return pl.BlockSpec(shape, lambda i: (0,) * len(shape))


def _enc_call(x, w1, b1, w2, b2, k):
    n, cin = x.shape
    hid = w1.shape[1]
    emb = w2.shape[1]
    r = 1000
    return pl.pallas_call(
        functools.partial(_enc_body, k=k),
        grid=(n // r,),
        in_specs=[
            pl.BlockSpec((r, cin), lambda i: (i, 0)),
            _full_spec((cin, hid)), _full_spec((1, hid)),
            _full_spec((hid, emb)), _full_spec((1, emb)),
        ],
        out_specs=pl.BlockSpec((r, emb), lambda i: (i, 0)),
        out_shape=jax.ShapeDtypeStruct((n, emb), jnp.float32),
    )(x, w1, b1, w2, b2)


def _gine_mlp_call(h, p0, p1, w1, b1, w2, b2, k):
    n, emb = h.shape
    hid = w1.shape[1]
    r = 1000
    blk = pl.BlockSpec((r, emb), lambda i: (i, 0))
    return pl.pallas_call(
        functools.partial(_gine_mlp_body, k=k),
        grid=(n // r,),
        in_specs=[
            blk, blk, blk,
            _full_spec((emb, hid)), _full_spec((1, hid)),
            _full_spec((hid, emb)), _full_spec((1, emb)),
        ],
        out_specs=blk,
        out_shape=jax.ShapeDtypeStruct((n, emb), jnp.float32),
    )(h, p0, p1, w1, b1, w2, b2)


def _post_call(h, w, b, k):
    n, emb = h.shape
    cout = w.shape[1]
    r = 1000
    return pl.pallas_call(
        functools.partial(_post_body, k=k),
        grid=(n // r,),
        in_specs=[
            pl.BlockSpec((r, emb), lambda i: (i, 0)),
            _full_spec((emb, cout)), _full_spec((1, cout)),
        ],
        out_specs=pl.BlockSpec((r, cout), lambda i: (i, 0)),
        out_shape=jax.ShapeDtypeStruct((n, cout), jnp.float32),
    )(h, w, b)


def _edge_call(ew, we1, be1, we2, be2):
    e, ed = ew.shape
    emb = we1.shape[1]
    r = 4000
    return pl.pallas_call(
        _edge_body,
        grid=(e // r,),
        in_specs=[
            pl.BlockSpec((r, ed), lambda i: (i, 0)),
            _full_spec((ed, emb)), _full_spec((1, emb)),
            _full_spec((ed, emb)), _full_spec((1, emb)),
        ],
        out_specs=[pl.BlockSpec((r, emb), lambda i: (i, 0))] * 2,
        out_shape=[jax.ShapeDtypeStruct((e, emb), jnp.float32)] * 2,
    )(ew, we1, be1, we2, be2)


def _make_seg_kernel(n, e, emb):
    nchunks_total = e // _CHUNK
    chunks_per_core = nchunks_total // _NC
    base_ch = chunks_per_core // _NS
    extra_ch = chunks_per_core - base_ch * _NS
    zrows = (n // _NS) & ~7
    tail = n - _NS * zrows
    nvec = emb // 16
    mesh = plsc.VectorSubcoreMesh(core_axis_name="c", subcore_axis_name="s")

    @functools.partial(
        pl.kernel,
        out_type=jax.ShapeDtypeStruct((_NC, n, emb), jnp.float32),
        mesh=mesh,
        compiler_params=pltpu.CompilerParams(use_tc_tiling_on_sc=False),
        scratch_types=[
            pltpu.VMEM((2, _CHUNK), jnp.int32),
            pltpu.VMEM((2, _CHUNK), jnp.int32),
            pltpu.VMEM((2, _CHUNK, emb), jnp.float32),
            pltpu.VMEM((2, _CHUNK, emb), jnp.float32),
            pltpu.VMEM((zrows, emb), jnp.float32),
            pltpu.VMEM_SHARED((n, emb), jnp.float32),
            pltpu.SemaphoreType.DMA,
            pltpu.SemaphoreType.DMA,
            pltpu.SemaphoreType.DMA,
            pltpu.SemaphoreType.DMA,
        ],
    )
    def seg(h_hbm, e_hbm, src_hbm, dst_hbm, out_hbm,
            src_v, dst_v, rows_v, e_v, z_v, aggr_sh,
            sem_ge0, sem_ge1, sem_s0, sem_s1):
        c = lax.axis_index("c")
        s = lax.axis_index("s")
        sem_ge = (sem_ge0, sem_ge1)
        sem_s = (sem_s0, sem_s1)

        def zbody(r2, _):
            for v in range(nvec):
                z_v[r2, pl.ds(v * 16, 16)] = jnp.zeros((16,), jnp.float32)
            return 0
        lax.fori_loop(0, zrows, zbody, 0)
        pltpu.sync_copy(z_v, aggr_sh.at[pl.ds(s * zrows, zrows)])

        @pl.when(s == _NS - 1)
        def _():
            pltpu.sync_copy(z_v.at[pl.ds(0, tail)],
                            aggr_sh.at[pl.ds(_NS * zrows, tail)])
        plsc.subcore_barrier()

        tbase = c * chunks_per_core + s * base_ch
        nch = base_ch + jnp.where(s == _NS - 1, extra_ch, 0)
        npairs = nch // 2

        def issue(j, b):
            base = (tbase + j) * _CHUNK
            pltpu.sync_copy(src_hbm.at[pl.ds(base, _CHUNK)], src_v.at[b])
            pltpu.sync_copy(dst_hbm.at[pl.ds(base, _CHUNK)], dst_v.at[b])
            pltpu.async_copy(h_hbm.at[src_v.at[b]], rows_v.at[b], sem_ge[b])
            pltpu.async_copy(e_hbm.at[pl.ds(base, _CHUNK)], e_v.at[b], sem_ge[b])

        def wait_ge(b):
            pltpu.make_async_copy(h_hbm.at[src_v.at[b]], rows_v.at[b], sem_ge[b]).wait()
            pltpu.make_async_copy(e_hbm.at[pl.ds(0, _CHUNK)], e_v.at[b], sem_ge[b]).wait()

        def compute(b):
            @plsc.parallel_loop(0, _CHUNK, 1, unroll=8)
            def _(r2):
                for v in range(nvec):
                    sl = pl.ds(v * 16, 16)
                    rows_v[b, r2, sl] = jnp.maximum(
                        rows_v[b, r2, sl] + e_v[b, r2, sl], 0.0)

        def scatter(b):
            pltpu.async_copy(rows_v.at[b], aggr_sh.at[dst_v.at[b]], sem_s[b],
                             add=True)

        def wait_s(b):
            pltpu.make_async_copy(rows_v.at[b], aggr_sh.at[dst_v.at[b]],
                                  sem_s[b]).wait()

        issue(0, 0)

        def pair(p, _):
            j0 = 2 * p

            @pl.when(p > 0)
            def _():
                wait_s(1)
            issue(j0 + 1, 1)
            wait_ge(0)
            compute(0)
            scatter(0)

            @pl.when(j0 + 2 < nch)
            def _():
                wait_s(0)
                issue(j0 + 2, 0)
            wait_ge(1)
            compute(1)
            scatter(1)
            return 0
        lax.fori_loop(0, npairs, pair, 0)

        @pl.when(nch % 2 == 1)
        def _():
            wait_ge(0)
            compute(0)
            scatter(0)
        wait_s(0)
        wait_s(1)

        plsc.subcore_barrier()
        pltpu.sync_copy(aggr_sh.at[pl.ds(s * zrows, zrows)],
                        out_hbm.at[c, pl.ds(s * zrows, zrows)])

        @pl.when(s == _NS - 1)
        def _():
            pltpu.sync_copy(aggr_sh.at[pl.ds(_NS * zrows, tail)],
                            out_hbm.at[c, pl.ds(_NS * zrows, tail)])

    return seg


def kernel(x, edge_index, edge_weight,
           enc_W1, enc_b1, enc_W2, enc_b2,
           e1_We, e1_be, g1_W1, g1_b1, g1_W2, g1_b2,
           e2_We, e2_be, g2_W1, g2_b1, g2_W2, g2_b2,
           post_W, post_b):
    n = x.shape[0]
    e = edge_weight.shape[0]
    emb = enc_W2.shape[1]
    tk = emb // 2
    src = edge_index[0]
    dst = edge_index[1]

    h0 = _enc_call(x, enc_W1, enc_b1.reshape(1, -1), enc_W2,
                   enc_b2.reshape(1, -1), tk)
    e1, e2 = _edge_call(edge_weight, e1_We, e1_be.reshape(1, -1),
                        e2_We, e2_be.reshape(1, -1))
    seg = _make_seg_kernel(n, e, emb)

    p1 = seg(h0, e1, src, dst)
    h1 = _gine_mlp_call(h0, p1[0], p1[1], g1_W1, g1_b1.reshape(1, -1),
                        g1_W2, g1_b2.reshape(1, -1), tk)
    p2 = seg(h1, e2, src, dst)
    h2 = _gine_mlp_call(h1, p2[0], p2[1], g2_W1, g2_b1.reshape(1, -1),
                        g2_W2, g2_b2.reshape(1, -1), tk)
    return _post_call(h2, post_W, post_b.reshape(1, -1), post_W.shape[1] // 2)


# speedup vs baseline: 18.2830x; 1.0365x over previous
import functools

import jax
import jax.numpy as jnp
from jax import lax
from jax.experimental import pallas as pl
from jax.experimental.pallas import tpu as pltpu
from jax.experimental.pallas import tpu_sc as plsc

_NC = 2
_NS = 16
_CHUNK = 128
_BISECT_ITERS = 32


def _topk_mask_block(y, k):
    kf = jnp.float32(k)
    yt = y.T
    b = lax.bitcast_convert_type(yt, jnp.int32)
    key = jnp.where(b < 0, b ^ jnp.int32(0x7FFFFFFF), b)
    r = yt.shape[1]
    lo0 = jnp.full((1, r), -(2 ** 31), jnp.int32)
    hi0 = jnp.full((1, r), 2 ** 31 - 1, jnp.int32)

    def body(_, carry):
        lo, hi = carry
        mid = (lo & hi) + ((lo ^ hi) >> 1)
        cnt = jnp.sum((key > mid).astype(jnp.float32), axis=0, keepdims=True)
        pred = cnt >= kf
        return jnp.where(pred, mid, lo), jnp.where(pred, hi, mid)

    lo, _ = lax.fori_loop(0, _BISECT_ITERS, body, (lo0, hi0))
    return jnp.where(key > lo, jnp.maximum(yt, 0.0), 0.0).T


def _leaky(t):
    return jnp.where(t >= 0.0, t, 0.01 * t)


def _enc_body(x_ref, w1_ref, b1_ref, w2_ref, b2_ref, o_ref, *, k):
    h = jnp.dot(x_ref[...], w1_ref[...], preferred_element_type=jnp.float32)
    h = _leaky(h + b1_ref[...])
    y = jnp.dot(h, w2_ref[...], preferred_element_type=jnp.float32) + b2_ref[...]
    o_ref[...] = _topk_mask_block(y, k)


def _gine_mlp_body(h_ref, p0_ref, p1_ref, w1_ref, b1_ref, w2_ref, b2_ref,
                   o_ref, *, k):
    g = h_ref[...] + p0_ref[...] + p1_ref[...]
    t = jnp.dot(g, w1_ref[...], preferred_element_type=jnp.float32)
    t = _leaky(t + b1_ref[...])
    y = jnp.dot(t, w2_ref[...], preferred_element_type=jnp.float32) + b2_ref[...]
    o_ref[...] = h_ref[...] + _topk_mask_block(y, k)


def _gine_post_body(h_ref, p0_ref, p1_ref, w1_ref, b1_ref, w2_ref, b2_ref,
                    pw_ref, pb_ref, o_ref, *, k, kp):
    g = h_ref[...] + p0_ref[...] + p1_ref[...]
    t = jnp.dot(g, w1_ref[...], preferred_element_type=jnp.float32)
    t = _leaky(t + b1_ref[...])
    y = jnp.dot(t, w2_ref[...], preferred_element_type=jnp.float32) + b2_ref[...]
    h2 = h_ref[...] + _topk_mask_block(y, k)
    yp = jnp.dot(h2, pw_ref[...], preferred_element_type=jnp.float32) + pb_ref[...]
    o_ref[...] = _topk_mask_block(yp, kp)


def _edge_body(ew_ref, we1_ref, be1_ref, we2_ref, be2_ref, e1_ref, e2_ref):
    ew = ew_ref[...]
    e1_ref[...] = jnp.dot(ew, we1_ref[...], preferred_element_type=jnp.float32) + be1_ref[...]
    e2_ref[...] = jnp.dot(ew, we2_ref[...], preferred_element_type=jnp.float32) + be2_ref[...]


def _full_spec(shape):
    return pl.BlockSpec(shape, lambda i: (0,) * len(shape))


def _enc_call(x, w1, b1, w2, b2, k):
    n, cin = x.shape
    hid = w1.shape[1]
    emb = w2.shape[1]
    r = 1000
    return pl.pallas_call(
        functools.partial(_enc_body, k=k),
        grid=(n // r,),
        in_specs=[
            pl.BlockSpec((r, cin), lambda i: (i, 0)),
            _full_spec((cin, hid)), _full_spec((1, hid)),
            _full_spec((hid, emb)), _full_spec((1, emb)),
        ],
        out_specs=pl.BlockSpec((r, emb), lambda i: (i, 0)),
        out_shape=jax.ShapeDtypeStruct((n, emb), jnp.float32),
    )(x, w1, b1, w2, b2)


def _gine_mlp_call(h, p0, p1, w1, b1, w2, b2, k):
    n, emb = h.shape
    hid = w1.shape[1]
    r = 1000
    blk = pl.BlockSpec((r, emb), lambda i: (i, 0))
    return pl.pallas_call(
        functools.partial(_gine_mlp_body, k=k),
        grid=(n // r,),
        in_specs=[
            blk, blk, blk,
            _full_spec((emb, hid)), _full_spec((1, hid)),
            _full_spec((hid, emb)), _full_spec((1, emb)),
        ],
        out_specs=blk,
        out_shape=jax.ShapeDtypeStruct((n, emb), jnp.float32),
    )(h, p0, p1, w1, b1, w2, b2)


def _gine_post_call(h, p0, p1, w1, b1, w2, b2, pw, pb, k, kp):
    n, emb = h.shape
    hid = w1.shape[1]
    cout = pw.shape[1]
    r = 1000
    blk = pl.BlockSpec((r, emb), lambda i: (i, 0))
    return pl.pallas_call(
        functools.partial(_gine_post_body, k=k, kp=kp),
        grid=(n // r,),
        in_specs=[
            blk, blk, blk,
            _full_spec((emb, hid)), _full_spec((1, hid)),
            _full_spec((hid, emb)), _full_spec((1, emb)),
            _full_spec((emb, cout)), _full_spec((1, cout)),
        ],
        out_specs=pl.BlockSpec((r, cout), lambda i: (i, 0)),
        out_shape=jax.ShapeDtypeStruct((n, cout), jnp.float32),
    )(h, p0, p1, w1, b1, w2, b2, pw, pb)


def _edge_call(ew, we1, be1, we2, be2):
    e, ed = ew.shape
    emb = we1.shape[1]
    r = 4000
    return pl.pallas_call(
        _edge_body,
        grid=(e // r,),
        in_specs=[
            pl.BlockSpec((r, ed), lambda i: (i, 0)),
            _full_spec((ed, emb)), _full_spec((1, emb)),
            _full_spec((ed, emb)), _full_spec((1, emb)),
        ],
        out_specs=[pl.BlockSpec((r, emb), lambda i: (i, 0))] * 2,
        out_shape=[jax.ShapeDtypeStruct((e, emb), jnp.float32)] * 2,
    )(ew, we1, be1, we2, be2)


def _make_seg_kernel(n, e, emb):
    nchunks_total = e // _CHUNK
    chunks_per_core = nchunks_total // _NC
    base_ch = chunks_per_core // _NS
    extra_ch = chunks_per_core - base_ch * _NS
    zrows = (n // _NS) & ~7
    tail = n - _NS * zrows
    nvec = emb // 16
    mesh = plsc.VectorSubcoreMesh(core_axis_name="c", subcore_axis_name="s")

    @functools.partial(
        pl.kernel,
        out_type=jax.ShapeDtypeStruct((_NC, n, emb), jnp.float32),
        mesh=mesh,
        compiler_params=pltpu.CompilerParams(use_tc_tiling_on_sc=False),
        scratch_types=[
            pltpu.VMEM((2, _CHUNK), jnp.int32),
            pltpu.VMEM((2, _CHUNK), jnp.int32),
            pltpu.VMEM((2, _CHUNK, emb), jnp.float32),
            pltpu.VMEM((2, _CHUNK, emb), jnp.float32),
            pltpu.VMEM_SHARED((n, emb), jnp.float32),
            pltpu.SemaphoreType.DMA,
            pltpu.SemaphoreType.DMA,
            pltpu.SemaphoreType.DMA,
            pltpu.SemaphoreType.DMA,
        ],
    )
    def seg(h_hbm, e_hbm, src_hbm, dst_hbm, zeros_hbm, out_hbm,
            src_v, dst_v, rows_v, e_v, aggr_sh,
            sem_ge0, sem_ge1, sem_s0, sem_s1):
        c = lax.axis_index("c")
        s = lax.axis_index("s")
        sem_ge = (sem_ge0, sem_ge1)
        sem_s = (sem_s0, sem_s1)

        pltpu.sync_copy(zeros_hbm.at[pl.ds(s * zrows, zrows)],
                        aggr_sh.at[pl.ds(s * zrows, zrows)])

        @pl.when(s == _NS - 1)
        def _():
            pltpu.sync_copy(zeros_hbm.at[pl.ds(_NS * zrows, tail)],
                            aggr_sh.at[pl.ds(_NS * zrows, tail)])
        plsc.subcore_barrier()

        tbase = c * chunks_per_core + s * base_ch
        nch = base_ch + jnp.where(s == _NS - 1, extra_ch, 0)
        npairs = nch // 2

        def issue(j, b):
            base = (tbase + j) * _CHUNK
            pltpu.sync_copy(src_hbm.at[pl.ds(base, _CHUNK)], src_v.at[b])
            pltpu.sync_copy(dst_hbm.at[pl.ds(base, _CHUNK)], dst_v.at[b])
            pltpu.async_copy(h_hbm.at[src_v.at[b]], rows_v.at[b], sem_ge[b])
            pltpu.async_copy(e_hbm.at[pl.ds(base, _CHUNK)], e_v.at[b], sem_ge[b])

        def wait_ge(b):
            pltpu.make_async_copy(h_hbm.at[src_v.at[b]], rows_v.at[b], sem_ge[b]).wait()
            pltpu.make_async_copy(e_hbm.at[pl.ds(0, _CHUNK)], e_v.at[b], sem_ge[b]).wait()

        def compute(b):
            @plsc.parallel_loop(0, _CHUNK, 1, unroll=8)
            def _(r2):
                for v in range(nvec):
                    sl = pl.ds(v * 16, 16)
                    rows_v[b, r2, sl] = jnp.maximum(
                        rows_v[b, r2, sl] + e_v[b, r2, sl], 0.0)

        def scatter(b):
            pltpu.async_copy(rows_v.at[b], aggr_sh.at[dst_v.at[b]], sem_s[b],
                             add=True)

        def wait_s(b):
            pltpu.make_async_copy(rows_v.at[b], aggr_sh.at[dst_v.at[b]],
                                  sem_s[b]).wait()

        issue(0, 0)

        def pair(p, _):
            j0 = 2 * p

            @pl.when(p > 0)
            def _():
                wait_s(1)
            issue(j0 + 1, 1)
            wait_ge(0)
            compute(0)
            scatter(0)

            @pl.when(j0 + 2 < nch)
            def _():
                wait_s(0)
                issue(j0 + 2, 0)
            wait_ge(1)
            compute(1)
            scatter(1)
            return 0
        lax.fori_loop(0, npairs, pair, 0)

        @pl.when(nch % 2 == 1)
        def _():
            wait_ge(0)
            compute(0)
            scatter(0)
        wait_s(0)
        wait_s(1)

        plsc.subcore_barrier()
        pltpu.sync_copy(aggr_sh.at[pl.ds(s * zrows, zrows)],
                        out_hbm.at[c, pl.ds(s * zrows, zrows)])

        @pl.when(s == _NS - 1)
        def _():
            pltpu.sync_copy(aggr_sh.at[pl.ds(_NS * zrows, tail)],
                            out_hbm.at[c, pl.ds(_NS * zrows, tail)])

    return seg


def kernel(x, edge_index, edge_weight,
           enc_W1, enc_b1, enc_W2, enc_b2,
           e1_We, e1_be, g1_W1, g1_b1, g1_W2, g1_b2,
           e2_We, e2_be, g2_W1, g2_b1, g2_W2, g2_b2,
           post_W, post_b):
    n = x.shape[0]
    e = edge_weight.shape[0]
    emb = enc_W2.shape[1]
    tk = emb // 2
    src = edge_index[0]
    dst = edge_index[1]

    h0 = _enc_call(x, enc_W1, enc_b1.reshape(1, -1), enc_W2,
                   enc_b2.reshape(1, -1), tk)
    e1, e2 = _edge_call(edge_weight, e1_We, e1_be.reshape(1, -1),
                        e2_We, e2_be.reshape(1, -1))
    seg = _make_seg_kernel(n, e, emb)
    zeros = jnp.zeros((n, emb), jnp.float32)

    p1 = seg(h0, e1, src, dst, zeros)
    h1 = _gine_mlp_call(h0, p1[0], p1[1], g1_W1, g1_b1.reshape(1, -1),
                        g1_W2, g1_b2.reshape(1, -1), tk)
    p2 = seg(h1, e2, src, dst, zeros)
    return _gine_post_call(h1, p2[0], p2[1], g2_W1, g2_b1.reshape(1, -1),
                           g2_W2, g2_b2.reshape(1, -1), post_W,
                           post_b.reshape(1, -1), tk, post_W.shape[1] // 2)


# speedup vs baseline: 21.6109x; 1.1820x over previous
import functools

import jax
import jax.numpy as jnp
from jax import lax
from jax.experimental import pallas as pl
from jax.experimental.pallas import tpu as pltpu
from jax.experimental.pallas import tpu_sc as plsc

_NC = 2
_NS = 16
_CHUNK = 128
_BISECT_ITERS = 32


def _topk_mask_block(y, k):
    kf = jnp.float32(k)
    yt = y.T
    b = lax.bitcast_convert_type(yt, jnp.int32)
    key = jnp.where(b < 0, b ^ jnp.int32(0x7FFFFFFF), b)
    r = yt.shape[1]
    lo0 = jnp.full((1, r), -(2 ** 31), jnp.int32)
    hi0 = jnp.full((1, r), 2 ** 31 - 1, jnp.int32)

    def body(_, carry):
        lo, hi = carry
        mid = (lo & hi) + ((lo ^ hi) >> 1)
        cnt = jnp.sum((key > mid).astype(jnp.float32), axis=0, keepdims=True)
        pred = cnt >= kf
        return jnp.where(pred, mid, lo), jnp.where(pred, hi, mid)

    lo, _ = lax.fori_loop(0, _BISECT_ITERS, body, (lo0, hi0))
    return jnp.where(key > lo, jnp.maximum(yt, 0.0), 0.0).T


def _leaky(t):
    return jnp.where(t >= 0.0, t, 0.01 * t)


def _enc_edge_body(x_ref, w1_ref, b1_ref, w2_ref, b2_ref,
                   ew_ref, we1_ref, be1_ref, we2_ref, be2_ref,
                   h_ref, e1_ref, e2_ref, *, k):
    h = jnp.dot(x_ref[...], w1_ref[...], preferred_element_type=jnp.float32)
    h = _leaky(h + b1_ref[...])
    y = jnp.dot(h, w2_ref[...], preferred_element_type=jnp.float32) + b2_ref[...]
    h_ref[...] = _topk_mask_block(y, k)
    ew = ew_ref[...]
    e1_ref[...] = jnp.dot(ew, we1_ref[...], preferred_element_type=jnp.float32) + be1_ref[...]
    e2_ref[...] = jnp.dot(ew, we2_ref[...], preferred_element_type=jnp.float32) + be2_ref[...]


def _gine_mlp_body(h_ref, p_ref, w1_ref, b1_ref, w2_ref, b2_ref,
                   o_ref, *, k):
    g = h_ref[...] + p_ref[0] + p_ref[1]
    t = jnp.dot(g, w1_ref[...], preferred_element_type=jnp.float32)
    t = _leaky(t + b1_ref[...])
    y = jnp.dot(t, w2_ref[...], preferred_element_type=jnp.float32) + b2_ref[...]
    o_ref[...] = h_ref[...] + _topk_mask_block(y, k)


def _gine_post_body(h_ref, p_ref, w1_ref, b1_ref, w2_ref, b2_ref,
                    pw_ref, pb_ref, o_ref, *, k, kp):
    g = h_ref[...] + p_ref[0] + p_ref[1]
    t = jnp.dot(g, w1_ref[...], preferred_element_type=jnp.float32)
    t = _leaky(t + b1_ref[...])
    y = jnp.dot(t, w2_ref[...], preferred_element_type=jnp.float32) + b2_ref[...]
    h2 = h_ref[...] + _topk_mask_block(y, k)
    yp = jnp.dot(h2, pw_ref[...], preferred_element_type=jnp.float32) + pb_ref[...]
    o_ref[...] = _topk_mask_block(yp, kp)


def _full_spec(shape):
    return pl.BlockSpec(shape, lambda i: (0,) * len(shape))


def _enc_edge_call(x, w1, b1, w2, b2, ew, we1, be1, we2, be2, k):
    n, cin = x.shape
    hid = w1.shape[1]
    emb = w2.shape[1]
    e, ed = ew.shape
    r = 1000
    g = n // r
    re = e // g
    return pl.pallas_call(
        functools.partial(_enc_edge_body, k=k),
        grid=(g,),
        in_specs=[
            pl.BlockSpec((r, cin), lambda i: (i, 0)),
            _full_spec((cin, hid)), _full_spec((1, hid)),
            _full_spec((hid, emb)), _full_spec((1, emb)),
            pl.BlockSpec((re, ed), lambda i: (i, 0)),
            _full_spec((ed, emb)), _full_spec((1, emb)),
            _full_spec((ed, emb)), _full_spec((1, emb)),
        ],
        out_specs=[pl.BlockSpec((r, emb), lambda i: (i, 0)),
                   pl.BlockSpec((re, emb), lambda i: (i, 0)),
                   pl.BlockSpec((re, emb), lambda i: (i, 0))],
        out_shape=[jax.ShapeDtypeStruct((n, emb), jnp.float32),
                   jax.ShapeDtypeStruct((e, emb), jnp.float32),
                   jax.ShapeDtypeStruct((e, emb), jnp.float32)],
    )(x, w1, b1, w2, b2, ew, we1, be1, we2, be2)


def _gine_mlp_call(h, p, w1, b1, w2, b2, k):
    n, emb = h.shape
    hid = w1.shape[1]
    r = 1000
    blk = pl.BlockSpec((r, emb), lambda i: (i, 0))
    return pl.pallas_call(
        functools.partial(_gine_mlp_body, k=k),
        grid=(n // r,),
        in_specs=[
            blk, pl.BlockSpec((2, r, emb), lambda i: (0, i, 0)),
            _full_spec((emb, hid)), _full_spec((1, hid)),
            _full_spec((hid, emb)), _full_spec((1, emb)),
        ],
        out_specs=blk,
        out_shape=jax.ShapeDtypeStruct((n, emb), jnp.float32),
    )(h, p, w1, b1, w2, b2)


def _gine_post_call(h, p, w1, b1, w2, b2, pw, pb, k, kp):
    n, emb = h.shape
    hid = w1.shape[1]
    cout = pw.shape[1]
    r = 1000
    blk = pl.BlockSpec((r, emb), lambda i: (i, 0))
    return pl.pallas_call(
        functools.partial(_gine_post_body, k=k, kp=kp),
        grid=(n // r,),
        in_specs=[
            blk, pl.BlockSpec((2, r, emb), lambda i: (0, i, 0)),
            _full_spec((emb, hid)), _full_spec((1, hid)),
            _full_spec((hid, emb)), _full_spec((1, emb)),
            _full_spec((emb, cout)), _full_spec((1, cout)),
        ],
        out_specs=pl.BlockSpec((r, cout), lambda i: (i, 0)),
        out_shape=jax.ShapeDtypeStruct((n, cout), jnp.float32),
    )(h, p, w1, b1, w2, b2, pw, pb)


def _make_seg_kernel(n, e, emb):
    nchunks_total = e // _CHUNK
    chunks_per_core = nchunks_total // _NC
    base_ch = chunks_per_core // _NS
    extra_ch = chunks_per_core - base_ch * _NS
    zrows = (n // _NS) & ~7
    tail = n - _NS * zrows
    nvec = emb // 16
    mesh = plsc.VectorSubcoreMesh(core_axis_name="c", subcore_axis_name="s")

    @functools.partial(
        pl.kernel,
        out_type=jax.ShapeDtypeStruct((_NC, n, emb), jnp.float32),
        mesh=mesh,
        compiler_params=pltpu.CompilerParams(use_tc_tiling_on_sc=False),
        scratch_types=[
            pltpu.VMEM((base_ch + 1, _CHUNK), jnp.int32),
            pltpu.VMEM((base_ch + 1, _CHUNK), jnp.int32),
            pltpu.VMEM((2, _CHUNK, emb), jnp.float32),
            pltpu.VMEM((2, _CHUNK, emb), jnp.float32),
            pltpu.VMEM_SHARED((n, emb), jnp.float32),
            pltpu.SemaphoreType.DMA,
            pltpu.SemaphoreType.DMA,
            pltpu.SemaphoreType.DMA,
            pltpu.SemaphoreType.DMA,
        ],
    )
    def seg(h_hbm, e_hbm, src_hbm, dst_hbm, zeros_hbm, out_hbm,
            src_all, dst_all, rows_v, e_v, aggr_sh,
            sem_ge0, sem_ge1, sem_s0, sem_s1):
        c = lax.axis_index("c")
        s = lax.axis_index("s")
        sem_ge = (sem_ge0, sem_ge1)
        sem_s = (sem_s0, sem_s1)

        tbase = c * chunks_per_core + s * base_ch
        nch = base_ch + jnp.where(s == _NS - 1, extra_ch, 0)
        npairs = nch // 2

        pltpu.sync_copy(src_hbm.at[pl.ds(tbase, base_ch + 1)], src_all)
        pltpu.sync_copy(dst_hbm.at[pl.ds(tbase, base_ch + 1)], dst_all)

        pltpu.sync_copy(zeros_hbm.at[pl.ds(s * zrows, zrows)],
                        aggr_sh.at[pl.ds(s * zrows, zrows)])

        @pl.when(s == _NS - 1)
        def _():
            pltpu.sync_copy(zeros_hbm.at[pl.ds(_NS * zrows, tail)],
                            aggr_sh.at[pl.ds(_NS * zrows, tail)])
        plsc.subcore_barrier()

        def issue(j, b):
            base = (tbase + j) * _CHUNK
            pltpu.async_copy(h_hbm.at[src_all.at[j]], rows_v.at[b], sem_ge[b])
            pltpu.async_copy(e_hbm.at[pl.ds(base, _CHUNK)], e_v.at[b], sem_ge[b])

        def wait_ge(b):
            pltpu.make_async_copy(h_hbm.at[src_all.at[0]], rows_v.at[b], sem_ge[b]).wait()
            pltpu.make_async_copy(e_hbm.at[pl.ds(0, _CHUNK)], e_v.at[b], sem_ge[b]).wait()

        def compute(b):
            @plsc.parallel_loop(0, _CHUNK, 1, unroll=8)
            def _(r2):
                for v in range(nvec):
                    sl = pl.ds(v * 16, 16)
                    rows_v[b, r2, sl] = jnp.maximum(
                        rows_v[b, r2, sl] + e_v[b, r2, sl], 0.0)

        def scatter(j, b):
            pltpu.async_copy(rows_v.at[b], aggr_sh.at[dst_all.at[j]], sem_s[b],
                             add=True)

        def wait_s(b):
            pltpu.make_async_copy(rows_v.at[b], aggr_sh.at[dst_all.at[0]],
                                  sem_s[b]).wait()

        issue(0, 0)

        def pair(p, _):
            j0 = 2 * p

            @pl.when(p > 0)
            def _():
                wait_s(1)
            issue(j0 + 1, 1)
            wait_ge(0)
            compute(0)
            scatter(j0, 0)

            @pl.when(j0 + 2 < nch)
            def _():
                wait_s(0)
                issue(j0 + 2, 0)
            wait_ge(1)
            compute(1)
            scatter(j0 + 1, 1)
            return 0
        lax.fori_loop(0, npairs, pair, 0)

        @pl.when(nch % 2 == 1)
        def _():
            wait_ge(0)
            compute(0)
            scatter(nch - 1, 0)
        wait_s(0)
        wait_s(1)

        plsc.subcore_barrier()
        pltpu.sync_copy(aggr_sh.at[pl.ds(s * zrows, zrows)],
                        out_hbm.at[c, pl.ds(s * zrows, zrows)])

        @pl.when(s == _NS - 1)
        def _():
            pltpu.sync_copy(aggr_sh.at[pl.ds(_NS * zrows, tail)],
                            out_hbm.at[c, pl.ds(_NS * zrows, tail)])

    return seg


def kernel(x, edge_index, edge_weight,
           enc_W1, enc_b1, enc_W2, enc_b2,
           e1_We, e1_be, g1_W1, g1_b1, g1_W2, g1_b2,
           e2_We, e2_be, g2_W1, g2_b1, g2_W2, g2_b2,
           post_W, post_b):
    n = x.shape[0]
    e = edge_weight.shape[0]
    emb = enc_W2.shape[1]
    tk = emb // 2
    src = edge_index[0].reshape(-1, _CHUNK)
    dst = edge_index[1].reshape(-1, _CHUNK)

    h0, e1, e2 = _enc_edge_call(x, enc_W1, enc_b1.reshape(1, -1), enc_W2,
                                enc_b2.reshape(1, -1), edge_weight,
                                e1_We, e1_be.reshape(1, -1),
                                e2_We, e2_be.reshape(1, -1), tk)
    seg = _make_seg_kernel(n, e, emb)
    zeros = jnp.zeros((n, emb), jnp.float32)

    p1 = seg(h0, e1, src, dst, zeros)
    h1 = _gine_mlp_call(h0, p1, g1_W1, g1_b1.reshape(1, -1),
                        g1_W2, g1_b2.reshape(1, -1), tk)
    p2 = seg(h1, e2, src, dst, zeros)
    return _gine_post_call(h1, p2, g2_W1, g2_b1.reshape(1, -1),
                           g2_W2, g2_b2.reshape(1, -1), post_W,
                           post_b.reshape(1, -1), tk, post_W.shape[1] // 2)


# speedup vs baseline: 21.6199x; 1.0004x over previous
import functools

import jax
import jax.numpy as jnp
from jax import lax
from jax.experimental import pallas as pl
from jax.experimental.pallas import tpu as pltpu
from jax.experimental.pallas import tpu_sc as plsc

_NC = 2
_NS = 16
_CHUNK = 128
_BISECT_ITERS = 32


def _topk_mask_block(y, k):
    kf = jnp.float32(k)
    yt = y.T
    b = lax.bitcast_convert_type(yt, jnp.int32)
    key = jnp.where(b < 0, b ^ jnp.int32(0x7FFFFFFF), b)
    r = yt.shape[1]
    lo0 = jnp.full((1, r), -(2 ** 31), jnp.int32)
    hi0 = jnp.full((1, r), 2 ** 31 - 1, jnp.int32)

    def body(_, carry):
        lo, hi = carry
        mid = (lo & hi) + ((lo ^ hi) >> 1)
        cnt = jnp.sum((key > mid).astype(jnp.float32), axis=0, keepdims=True)
        pred = cnt >= kf
        return jnp.where(pred, mid, lo), jnp.where(pred, hi, mid)

    lo, _ = lax.fori_loop(0, _BISECT_ITERS, body, (lo0, hi0))
    return jnp.where(key > lo, jnp.maximum(yt, 0.0), 0.0).T


def _leaky(t):
    return jnp.where(t >= 0.0, t, 0.01 * t)


def _enc_edge_body(x_ref, w1_ref, b1_ref, w2_ref, b2_ref,
                   ew_ref, we1_ref, be1_ref, we2_ref, be2_ref,
                   h_ref, e1_ref, e2_ref, *, k):
    h = jnp.dot(x_ref[...], w1_ref[...], preferred_element_type=jnp.float32)
    h = _leaky(h + b1_ref[...])
    y = jnp.dot(h, w2_ref[...], preferred_element_type=jnp.float32) + b2_ref[...]
    h_ref[...] = _topk_mask_block(y, k)
    ew = ew_ref[...]
    e1_ref[...] = jnp.dot(ew, we1_ref[...], preferred_element_type=jnp.float32) + be1_ref[...]
    e2_ref[...] = jnp.dot(ew, we2_ref[...], preferred_element_type=jnp.float32) + be2_ref[...]


def _gine_mlp_body(h_ref, p_ref, w1_ref, b1_ref, w2_ref, b2_ref,
                   o_ref, *, k):
    g = h_ref[...] + p_ref[0] + p_ref[1]
    t = jnp.dot(g, w1_ref[...], preferred_element_type=jnp.float32)
    t = _leaky(t + b1_ref[...])
    y = jnp.dot(t, w2_ref[...], preferred_element_type=jnp.float32) + b2_ref[...]
    o_ref[...] = h_ref[...] + _topk_mask_block(y, k)


def _gine_post_body(h_ref, p_ref, w1_ref, b1_ref, w2_ref, b2_ref,
                    pw_ref, pb_ref, o_ref, *, k, kp):
    g = h_ref[...] + p_ref[0] + p_ref[1]
    t = jnp.dot(g, w1_ref[...], preferred_element_type=jnp.float32)
    t = _leaky(t + b1_ref[...])
    y = jnp.dot(t, w2_ref[...], preferred_element_type=jnp.float32) + b2_ref[...]
    h2 = h_ref[...] + _topk_mask_block(y, k)
    yp = jnp.dot(h2, pw_ref[...], preferred_element_type=jnp.float32) + pb_ref[...]
    o_ref[...] = _topk_mask_block(yp, kp)


def _full_spec(shape):
    return pl.BlockSpec(shape, lambda i: (0,) * len(shape))


def _enc_edge_call(x, w1, b1, w2, b2, ew, we1, be1, we2, be2, k):
    n, cin = x.shape
    hid = w1.shape[1]
    emb = w2.shape[1]
    e, ed = ew.shape
    r = 1000
    g = n // r
    re = e // g
    return pl.pallas_call(
        functools.partial(_enc_edge_body, k=k),
        grid=(g,),
        in_specs=[
            pl.BlockSpec((r, cin), lambda i: (i, 0)),
            _full_spec((cin, hid)), _full_spec((1, hid)),
            _full_spec((hid, emb)), _full_spec((1, emb)),
            pl.BlockSpec((re, ed), lambda i: (i, 0)),
            _full_spec((ed, emb)), _full_spec((1, emb)),
            _full_spec((ed, emb)), _full_spec((1, emb)),
        ],
        out_specs=[pl.BlockSpec((r, emb), lambda i: (i, 0)),
                   pl.BlockSpec((re, emb), lambda i: (i, 0)),
                   pl.BlockSpec((re, emb), lambda i: (i, 0))],
        out_shape=[jax.ShapeDtypeStruct((n, emb), jnp.float32),
                   jax.ShapeDtypeStruct((e, emb), jnp.float32),
                   jax.ShapeDtypeStruct((e, emb), jnp.float32)],
    )(x, w1, b1, w2, b2, ew, we1, be1, we2, be2)


def _gine_mlp_call(h, p, w1, b1, w2, b2, k):
    n, emb = h.shape
    hid = w1.shape[1]
    r = 1000
    blk = pl.BlockSpec((r, emb), lambda i: (i, 0))
    return pl.pallas_call(
        functools.partial(_gine_mlp_body, k=k),
        grid=(n // r,),
        in_specs=[
            blk, pl.BlockSpec((2, r, emb), lambda i: (0, i, 0)),
            _full_spec((emb, hid)), _full_spec((1, hid)),
            _full_spec((hid, emb)), _full_spec((1, emb)),
        ],
        out_specs=blk,
        out_shape=jax.ShapeDtypeStruct((n, emb), jnp.float32),
    )(h, p, w1, b1, w2, b2)


def _gine_post_call(h, p, w1, b1, w2, b2, pw, pb, k, kp):
    n, emb = h.shape
    hid = w1.shape[1]
    cout = pw.shape[1]
    r = 1000
    blk = pl.BlockSpec((r, emb), lambda i: (i, 0))
    return pl.pallas_call(
        functools.partial(_gine_post_body, k=k, kp=kp),
        grid=(n // r,),
        in_specs=[
            blk, pl.BlockSpec((2, r, emb), lambda i: (0, i, 0)),
            _full_spec((emb, hid)), _full_spec((1, hid)),
            _full_spec((hid, emb)), _full_spec((1, emb)),
            _full_spec((emb, cout)), _full_spec((1, cout)),
        ],
        out_specs=pl.BlockSpec((r, cout), lambda i: (i, 0)),
        out_shape=jax.ShapeDtypeStruct((n, cout), jnp.float32),
    )(h, p, w1, b1, w2, b2, pw, pb)


def _make_seg_kernel(n, e, emb):
    nchunks_total = e // _CHUNK
    chunks_per_core = nchunks_total // _NC
    base_ch = chunks_per_core // _NS
    extra_ch = chunks_per_core - base_ch * _NS
    zrows = (n // _NS) & ~7
    tail = n - _NS * zrows
    nvec = emb // 16
    mesh = plsc.VectorSubcoreMesh(core_axis_name="c", subcore_axis_name="s")

    @functools.partial(
        pl.kernel,
        out_type=jax.ShapeDtypeStruct((_NC, n, emb), jnp.float32),
        mesh=mesh,
        compiler_params=pltpu.CompilerParams(use_tc_tiling_on_sc=False,
                                             skip_device_barrier=True),
        scratch_types=[
            pltpu.VMEM((base_ch + 1, _CHUNK), jnp.int32),
            pltpu.VMEM((base_ch + 1, _CHUNK), jnp.int32),
            pltpu.VMEM((2, _CHUNK, emb), jnp.float32),
            pltpu.VMEM((2, _CHUNK, emb), jnp.float32),
            pltpu.VMEM_SHARED((n, emb), jnp.float32),
            pltpu.SemaphoreType.DMA,
            pltpu.SemaphoreType.DMA,
            pltpu.SemaphoreType.DMA,
            pltpu.SemaphoreType.DMA,
        ],
    )
    def seg(h_hbm, e_hbm, src_hbm, dst_hbm, zeros_hbm, out_hbm,
            src_all, dst_all, rows_v, e_v, aggr_sh,
            sem_ge0, sem_ge1, sem_s0, sem_s1):
        c = lax.axis_index("c")
        s = lax.axis_index("s")
        sem_ge = (sem_ge0, sem_ge1)
        sem_s = (sem_s0, sem_s1)

        tbase = c * chunks_per_core + s * base_ch
        nch = base_ch + jnp.where(s == _NS - 1, extra_ch, 0)
        npairs = nch // 2

        pltpu.sync_copy(src_hbm.at[pl.ds(tbase, base_ch + 1)], src_all)
        pltpu.sync_copy(dst_hbm.at[pl.ds(tbase, base_ch + 1)], dst_all)

        pltpu.sync_copy(zeros_hbm.at[pl.ds(s * zrows, zrows)],
                        aggr_sh.at[pl.ds(s * zrows, zrows)])

        @pl.when(s == _NS - 1)
        def _():
            pltpu.sync_copy(zeros_hbm.at[pl.ds(_NS * zrows, tail)],
                            aggr_sh.at[pl.ds(_NS * zrows, tail)])
        plsc.subcore_barrier()

        def issue(j, b):
            base = (tbase + j) * _CHUNK
            pltpu.async_copy(h_hbm.at[src_all.at[j]], rows_v.at[b], sem_ge[b])
            pltpu.async_copy(e_hbm.at[pl.ds(base, _CHUNK)], e_v.at[b], sem_ge[b])

        def wait_ge(b):
            pltpu.make_async_copy(h_hbm.at[src_all.at[0]], rows_v.at[b], sem_ge[b]).wait()
            pltpu.make_async_copy(e_hbm.at[pl.ds(0, _CHUNK)], e_v.at[b], sem_ge[b]).wait()

        def compute(b):
            @plsc.parallel_loop(0, _CHUNK, 1, unroll=8)
            def _(r2):
                for v in range(nvec):
                    sl = pl.ds(v * 16, 16)
                    rows_v[b, r2, sl] = jnp.maximum(
                        rows_v[b, r2, sl] + e_v[b, r2, sl], 0.0)

        def scatter(j, b):
            pltpu.async_copy(rows_v.at[b], aggr_sh.at[dst_all.at[j]], sem_s[b],
                             add=True)

        def wait_s(b):
            pltpu.make_async_copy(rows_v.at[b], aggr_sh.at[dst_all.at[0]],
                                  sem_s[b]).wait()

        issue(0, 0)

        def pair(p, _):
            j0 = 2 * p

            @pl.when(p > 0)
            def _():
                wait_s(1)
            issue(j0 + 1, 1)
            wait_ge(0)
            compute(0)
            scatter(j0, 0)

            @pl.when(j0 + 2 < nch)
            def _():
                wait_s(0)
                issue(j0 + 2, 0)
            wait_ge(1)
            compute(1)
            scatter(j0 + 1, 1)
            return 0
        lax.fori_loop(0, npairs, pair, 0)

        @pl.when(nch % 2 == 1)
        def _():
            wait_ge(0)
            compute(0)
            scatter(nch - 1, 0)
        wait_s(0)
        wait_s(1)

        plsc.subcore_barrier()
        pltpu.sync_copy(aggr_sh.at[pl.ds(s * zrows, zrows)],
                        out_hbm.at[c, pl.ds(s * zrows, zrows)])

        @pl.when(s == _NS - 1)
        def _():
            pltpu.sync_copy(aggr_sh.at[pl.ds(_NS * zrows, tail)],
                            out_hbm.at[c, pl.ds(_NS * zrows, tail)])

    return seg


def kernel(x, edge_index, edge_weight,
           enc_W1, enc_b1, enc_W2, enc_b2,
           e1_We, e1_be, g1_W1, g1_b1, g1_W2, g1_b2,
           e2_We, e2_be, g2_W1, g2_b1, g2_W2, g2_b2,
           post_W, post_b):
    n = x.shape[0]
    e = edge_weight.shape[0]
    emb = enc_W2.shape[1]
    tk = emb // 2
    src = edge_index[0].reshape(-1, _CHUNK)
    dst = edge_index[1].reshape(-1, _CHUNK)

    h0, e1, e2 = _enc_edge_call(x, enc_W1, enc_b1.reshape(1, -1), enc_W2,
                                enc_b2.reshape(1, -1), edge_weight,
                                e1_We, e1_be.reshape(1, -1),
                                e2_We, e2_be.reshape(1, -1), tk)
    seg = _make_seg_kernel(n, e, emb)
    zeros = jnp.zeros((n, emb), jnp.float32)

    p1 = seg(h0, e1, src, dst, zeros)
    h1 = _gine_mlp_call(h0, p1, g1_W1, g1_b1.reshape(1, -1),
                        g1_W2, g1_b2.reshape(1, -1), tk)
    p2 = seg(h1, e2, src, dst, zeros)
    return _gine_post_call(h1, p2, g2_W1, g2_b1.reshape(1, -1),
                           g2_W2, g2_b2.reshape(1, -1), post_W,
                           post_b.reshape(1, -1), tk, post_W.shape[1] // 2)
